# baseline (speedup 1.0000x reference)
import jax
import jax.numpy as jnp
from jax import lax
from jax.experimental import pallas as pl
from jax.experimental.pallas import tpu as pltpu
from jax.experimental.pallas import tpu_sc as plsc

N = 10000
E = 160000
D = 256
P = 5
NP = N * P
L = 16
NC, NS = 2, 16
NW = NC * NS

PW = 1568
PW_TAIL = NP - 31 * PW
FB = 112
ORG_ROWS = 624
ORG_CH = 208
EW = 4992

_STEPS = (8192, 4096, 2048, 1024, 512, 256, 128, 64, 32, 16, 8, 4, 2, 1)


GPAD = 12288


def _body(orgv, orge, pm, genq, outt, oute,
          pmv, goffs, sidx, esrc, edst, estg, zbuf, gbuf, sem, wsem):
    c = lax.axis_index("c")
    s = lax.axis_index("s")
    w = s * NC + c
    iota = lax.iota(jnp.int32, L)
    zrow16 = jnp.zeros((L,), jnp.float32)

    import contextlib
    ns = jax.named_scope
    _sc1 = ns("ph_orgcopy"); _sc1.__enter__()
    r0 = w * ORG_ROWS
    for h in range(0, ORG_ROWS, ORG_CH):
        pltpu.sync_copy(orgv.at[pl.ds(r0 + h, ORG_CH)], gbuf.at[pl.ds(0, ORG_CH)])
        pltpu.sync_copy(gbuf.at[pl.ds(0, ORG_CH)], outt.at[pl.ds(r0 + h, ORG_CH)])

    @pl.when(w == 0)
    def _():
        pltpu.sync_copy(orgv.at[pl.ds(NW * ORG_ROWS, 32)], gbuf.at[pl.ds(0, 32)])
        pltpu.sync_copy(gbuf.at[pl.ds(0, 32)], outt.at[pl.ds(NW * ORG_ROWS, 32)])

    _sc1.__exit__(None,None,None); _sc2 = ns("ph_edgecopy"); _sc2.__enter__()
    e0 = w * EW
    for r in (0, 1):
        pltpu.sync_copy(orge.at[r, pl.ds(e0, EW)], estg)
        pltpu.sync_copy(estg, oute.at[r, pl.ds(e0, EW)])

    @pl.when(w == 0)
    def _():
        for r in (0, 1):
            pltpu.sync_copy(orge.at[r, pl.ds(NW * EW, 256)], estg.at[pl.ds(0, 256)])
            pltpu.sync_copy(estg.at[pl.ds(0, 256)], oute.at[r, pl.ds(NW * EW, 256)])

    _sc2.__exit__(None,None,None); _sc3 = ns("ph_passa"); _sc3.__enter__()
    pltpu.sync_copy(pm, pmv)

    def pass_a(v, carry):
        x = pmv[pl.ds(v * L, L)]
        t = x.astype(jnp.int32)
        f = x - t.astype(jnp.float32)
        up = (f > 0.5) | ((f == 0.5) & (lax.rem(t, 2) != 0))
        k = jnp.clip(t + jnp.where(up, jnp.int32(1), jnp.int32(0)), 0, P)
        inc = plsc.cumsum(k)
        goffs[pl.ds(v * L, L)] = (inc - k) + carry
        return carry + inc[15]

    S = lax.fori_loop(0, N // L, pass_a, jnp.int32(0))

    padvec = jnp.full((L,), jnp.int32(2147483647))

    def pad_f(v, _):
        goffs[pl.ds(N + v * L, L)] = padvec
        return 0

    lax.fori_loop(0, (GPAD - N) // L, pad_f, 0)

    _sc3.__exit__(None,None,None); _sc4 = ns("ph_zero"); _sc4.__enter__()
    zcols = 128 // L

    def zero_row(ref, r):
        row = ref.at[r]
        for cc in range(zcols):
            row[pl.ds(cc * L, L)] = zrow16

    def zr_plain(r, _):
        zero_row(zbuf, r)
        return 0

    lax.fori_loop(0, 2 * FB, zr_plain, 0)

    _sc4.__exit__(None,None,None)
    def tile_work(pbase, npw):
        nv = npw // L

        def search_one(v):
            pvec = pbase + v * L + iota
            c0 = jnp.zeros((L,), jnp.int32)
            for step in _STEPS:
                g = plsc.load_gather(goffs, [c0 + (step - 1)])
                c0 = jnp.where(g <= pvec, c0 + step, c0)
            i = c0 - 1
            gi = plsc.load_gather(goffs, [i])
            j = pvec - gi
            valid = pvec < S
            m0 = ((i >> 3) * 10 + 2 * j) * 8 + (i & 7)
            m0 = jnp.where(valid, m0, 0)
            slot0 = v * 32 + (iota >> 3) * 16 + (iota & 7)
            plsc.store_scatter(sidx, [slot0], m0)
            plsc.store_scatter(sidx, [slot0 + 8], jnp.where(valid, m0 + 8, 0))
            ival = jnp.where(valid, i, -1)
            nval = jnp.where(valid, N + pvec, -1)
            ev = v * (2 * L) + iota * 2
            plsc.store_scatter(esrc, [ev], ival)
            plsc.store_scatter(esrc, [ev + 1], nval)
            plsc.store_scatter(edst, [ev], nval)
            plsc.store_scatter(edst, [ev + 1], ival)

        def search2(v, _):
            search_one(2 * v)
            search_one(2 * v + 1)
            return 0

        with ns("ph_search"):
            lax.fori_loop(0, nv // 2, search2, 0)
            if nv % 2:
                search_one(jnp.int32(nv - 1))

        pltpu.async_copy(esrc.at[pl.ds(0, 2 * npw)],
                         oute.at[0, pl.ds(E + 2 * pbase, 2 * npw)], wsem)
        pltpu.async_copy(edst.at[pl.ds(0, 2 * npw)],
                         oute.at[1, pl.ds(E + 2 * pbase, 2 * npw)], wsem)

        def do_batch(pstart, boff, bs):
            vb = jnp.clip(S - pstart, 0, bs)

            def gpath():
                cp0 = pltpu.async_copy(genq.at[sidx.at[pl.ds(boff, bs)]],
                                       gbuf.at[pl.ds(0, bs)], sem)
                cp1 = pltpu.async_copy(genq.at[sidx.at[pl.ds(boff + bs, bs)]],
                                       gbuf.at[pl.ds(bs, bs)], sem)
                cp0.wait()
                cp1.wait()

                @pl.when(vb < bs)
                def _():
                    def zr(r, _):
                        p_r = pstart + (r >> 4) * 8 + (r & 7)

                        @pl.when(p_r >= S)
                        def _():
                            zero_row(gbuf, r)
                        return 0
                    lax.fori_loop(0, 2 * bs, zr, 0)

                pltpu.sync_copy(gbuf.at[pl.ds(0, 2 * bs)],
                                outt.at[pl.ds(2 * N + 2 * pstart, 2 * bs)])

            def zpath():
                pltpu.async_copy(zbuf.at[pl.ds(0, 2 * bs)],
                                 outt.at[pl.ds(2 * N + 2 * pstart, 2 * bs)],
                                 wsem)

            lax.cond(vb > 0, gpath, zpath)

        def drain_batch(pstart, bs):
            vb = jnp.clip(S - pstart, 0, bs)

            @pl.when(vb == 0)
            def _():
                pltpu.make_async_copy(
                    zbuf.at[pl.ds(0, 2 * bs)],
                    outt.at[pl.ds(2 * N + 2 * pstart, 2 * bs)], wsem).wait()

        nb = npw // FB

        def batch(b, _):
            do_batch(pbase + b * FB, b * 2 * FB, FB)
            return 0

        with ns("ph_batches"):
            lax.fori_loop(0, nb, batch, 0)
            rem = npw - nb * FB
            if rem:
                do_batch(pbase + nb * FB, nb * 2 * FB, rem)

        rem = npw - nb * FB
        def batch_d(b, _):
            drain_batch(pbase + b * FB, FB)
            return 0

        lax.fori_loop(0, nb, batch_d, 0)
        if rem:
            drain_batch(pbase + nb * FB, rem)
        pltpu.make_async_copy(esrc.at[pl.ds(0, 2 * npw)],
                              oute.at[0, pl.ds(E + 2 * pbase, 2 * npw)],
                              wsem).wait()
        pltpu.make_async_copy(edst.at[pl.ds(0, 2 * npw)],
                              oute.at[1, pl.ds(E + 2 * pbase, 2 * npw)],
                              wsem).wait()

    lax.cond(w < NW - 1,
             lambda: tile_work(w * PW, PW),
             lambda: tile_work((NW - 1) * PW, PW_TAIL))


_mesh = plsc.VectorSubcoreMesh(core_axis_name="c", subcore_axis_name="s",
                               num_cores=NC, num_subcores=NS)

_sc_call = pl.kernel(
    _body,
    out_type=[
        jax.ShapeDtypeStruct((2 * (N + NP), 128), jnp.float32),
        jax.ShapeDtypeStruct((2, E + 2 * NP), jnp.int32),
    ],
    mesh=_mesh,
    scratch_types=[
        pltpu.VMEM((N,), jnp.float32),
        pltpu.VMEM((GPAD,), jnp.int32),
        pltpu.VMEM((2 * PW,), jnp.int32),
        pltpu.VMEM((2 * PW,), jnp.int32),
        pltpu.VMEM((2 * PW,), jnp.int32),
        pltpu.VMEM((EW,), jnp.int32),
        pltpu.VMEM((2 * FB, 128), jnp.float32),
        pltpu.VMEM((2 * FB, 128), jnp.float32),
        pltpu.SemaphoreType.DMA,
        pltpu.SemaphoreType.DMA,
    ],
    compiler_params=pltpu.CompilerParams(use_tc_tiling_on_sc=False,
                                         needs_layout_passes=False),
)


@jax.jit
def kernel(org_feats, org_edges, pred_missing, gen_feats):
    orgv = (org_feats.reshape(N // 8, 8, D // 128, 128)
            .transpose(0, 2, 1, 3).reshape(2 * N, 128))
    genq = (gen_feats.reshape(N // 8, 8, (P * D) // 128, 128)
            .transpose(0, 2, 1, 3).reshape(2 * NP, 128))
    outt, fill_edges = _sc_call(orgv, org_edges, pred_missing, genq)
    fill_feats = (outt.reshape((N + NP) // 8, 2, 8, 128)
                  .transpose(0, 2, 1, 3).reshape(N + NP, D))
    return (fill_feats, fill_edges)


# speedup vs baseline: 1.2469x; 1.2469x over previous
import jax
import jax.numpy as jnp
from jax import lax
from jax.experimental import pallas as pl
from jax.experimental.pallas import tpu as pltpu
from jax.experimental.pallas import tpu_sc as plsc

N = 10000
E = 160000
D = 256
P = 5
NP = N * P
L = 16
NC, NS = 2, 16
NW = NC * NS

PW = 1568
PW_TAIL = NP - 31 * PW
FB = 112
ORG_ROWS = 624
ORG_CH = 208
EW = 4992

_STEPS = (8192, 4096, 2048, 1024, 512, 256, 128, 64, 32, 16, 8, 4, 2, 1)


GPAD = 12288


def _body(orgv, orge, pm, genq, outt, oute,
          pmv, goffs, sidx, esrc, edst, estg, zbuf, gbuf, sem, wsem):
    c = lax.axis_index("c")
    s = lax.axis_index("s")
    w = s * NC + c
    iota = lax.iota(jnp.int32, L)
    zrow16 = jnp.zeros((L,), jnp.float32)

    import contextlib
    ns = jax.named_scope
    _sc1 = ns("ph_orgcopy"); _sc1.__enter__()
    r0 = w * ORG_ROWS
    for h in range(0, ORG_ROWS, ORG_CH):
        pltpu.sync_copy(orgv.at[pl.ds(r0 + h, ORG_CH)], gbuf.at[pl.ds(0, ORG_CH)])
        pltpu.sync_copy(gbuf.at[pl.ds(0, ORG_CH)], outt.at[pl.ds(r0 + h, ORG_CH)])

    @pl.when(w == 0)
    def _():
        pltpu.sync_copy(orgv.at[pl.ds(NW * ORG_ROWS, 32)], gbuf.at[pl.ds(0, 32)])
        pltpu.sync_copy(gbuf.at[pl.ds(0, 32)], outt.at[pl.ds(NW * ORG_ROWS, 32)])

    _sc1.__exit__(None,None,None); _sc2 = ns("ph_edgecopy"); _sc2.__enter__()
    e0 = w * EW
    for r in (0, 1):
        pltpu.sync_copy(orge.at[r, pl.ds(e0, EW)], estg)
        pltpu.sync_copy(estg, oute.at[r, pl.ds(e0, EW)])

    @pl.when(w == 0)
    def _():
        for r in (0, 1):
            pltpu.sync_copy(orge.at[r, pl.ds(NW * EW, 256)], estg.at[pl.ds(0, 256)])
            pltpu.sync_copy(estg.at[pl.ds(0, 256)], oute.at[r, pl.ds(NW * EW, 256)])

    _sc2.__exit__(None,None,None); _sc3 = ns("ph_passa"); _sc3.__enter__()
    pltpu.sync_copy(pm, pmv)

    def pass_a(v, carry):
        x = pmv[pl.ds(v * L, L)]
        t = x.astype(jnp.int32)
        f = x - t.astype(jnp.float32)
        up = (f > 0.5) | ((f == 0.5) & (lax.rem(t, 2) != 0))
        k = jnp.clip(t + jnp.where(up, jnp.int32(1), jnp.int32(0)), 0, P)
        inc = plsc.cumsum(k)
        goffs[pl.ds(v * L, L)] = (inc - k) + carry
        return carry + inc[15]

    S = lax.fori_loop(0, N // L, pass_a, jnp.int32(0))

    padvec = jnp.full((L,), jnp.int32(2147483647))

    def pad_f(v, _):
        goffs[pl.ds(N + v * L, L)] = padvec
        return 0

    lax.fori_loop(0, (GPAD - N) // L, pad_f, 0)

    _sc3.__exit__(None,None,None); _sc4 = ns("ph_zero"); _sc4.__enter__()
    zcols = 128 // L

    def zero_row(ref, r):
        row = ref.at[r]
        for cc in range(zcols):
            row[pl.ds(cc * L, L)] = zrow16

    def zr_plain(r, _):
        zero_row(zbuf, r)
        return 0

    lax.fori_loop(0, 2 * FB, zr_plain, 0)

    _sc4.__exit__(None,None,None)

    NBF = NP // FB
    BREM = NP - NBF * FB
    NBT = (NBF + 1 + NW - 1) // NW

    def search_batch(jj, bi, bs):
        nvb = bs // L
        base = jj * (2 * FB)
        pstart = bi * FB

        def search_one(v, _):
            pvec = pstart + v * L + iota
            c0 = jnp.zeros((L,), jnp.int32)
            for step in _STEPS:
                g = plsc.load_gather(goffs, [c0 + (step - 1)])
                c0 = jnp.where(g <= pvec, c0 + step, c0)
            i = c0 - 1
            gi = plsc.load_gather(goffs, [i])
            j = pvec - gi
            valid = pvec < S
            m0 = ((i >> 3) * 10 + 2 * j) * 8 + (i & 7)
            m0 = jnp.where(valid, m0, 0)
            slot0 = base + v * 32 + (iota >> 3) * 16 + (iota & 7)
            plsc.store_scatter(sidx, [slot0], m0)
            plsc.store_scatter(sidx, [slot0 + 8], jnp.where(valid, m0 + 8, 0))
            ival = jnp.where(valid, i, -1)
            nval = jnp.where(valid, N + pvec, -1)
            ev = base + v * (2 * L) + iota * 2
            plsc.store_scatter(esrc, [ev], ival)
            plsc.store_scatter(esrc, [ev + 1], nval)
            plsc.store_scatter(edst, [ev], nval)
            plsc.store_scatter(edst, [ev + 1], ival)
            return 0

        lax.fori_loop(0, nvb, search_one, 0)

    def do_batch(jj, bi, bs):
        search_batch(jj, bi, bs)
        base = jj * (2 * FB)
        pstart = bi * FB
        pltpu.async_copy(esrc.at[pl.ds(base, 2 * bs)],
                         oute.at[0, pl.ds(E + 2 * pstart, 2 * bs)], wsem)
        pltpu.async_copy(edst.at[pl.ds(base, 2 * bs)],
                         oute.at[1, pl.ds(E + 2 * pstart, 2 * bs)], wsem)
        vb = jnp.clip(S - pstart, 0, bs)

        def gpath():
            cp0 = pltpu.async_copy(genq.at[sidx.at[pl.ds(base, bs)]],
                                   gbuf.at[pl.ds(0, bs)], sem)
            cp1 = pltpu.async_copy(genq.at[sidx.at[pl.ds(base + bs, bs)]],
                                   gbuf.at[pl.ds(bs, bs)], sem)
            cp0.wait()
            cp1.wait()

            @pl.when(vb < bs)
            def _():
                def zr(r, _):
                    p_r = pstart + (r >> 4) * 8 + (r & 7)

                    @pl.when(p_r >= S)
                    def _():
                        zero_row(gbuf, r)
                    return 0
                lax.fori_loop(0, 2 * bs, zr, 0)

            pltpu.sync_copy(gbuf.at[pl.ds(0, 2 * bs)],
                            outt.at[pl.ds(2 * N + 2 * pstart, 2 * bs)])

        def zpath():
            pltpu.async_copy(zbuf.at[pl.ds(0, 2 * bs)],
                             outt.at[pl.ds(2 * N + 2 * pstart, 2 * bs)],
                             wsem)

        lax.cond(vb > 0, gpath, zpath)

    def drain_batch(jj, bi, bs):
        base = jj * (2 * FB)
        pstart = bi * FB
        pltpu.make_async_copy(esrc.at[pl.ds(base, 2 * bs)],
                              oute.at[0, pl.ds(E + 2 * pstart, 2 * bs)],
                              wsem).wait()
        pltpu.make_async_copy(edst.at[pl.ds(base, 2 * bs)],
                              oute.at[1, pl.ds(E + 2 * pstart, 2 * bs)],
                              wsem).wait()
        vb = jnp.clip(S - pstart, 0, bs)

        @pl.when(vb == 0)
        def _():
            pltpu.make_async_copy(
                zbuf.at[pl.ds(0, 2 * bs)],
                outt.at[pl.ds(2 * N + 2 * pstart, 2 * bs)], wsem).wait()

    def for_each_batch(fn):
        def body(jj, _):
            bi = w + NW * jj
            lax.cond(bi < NBF,
                     lambda: fn(jj, bi, FB),
                     lambda: lax.cond(bi == NBF,
                                      lambda: fn(jj, bi, BREM),
                                      lambda: None))
            return 0
        lax.fori_loop(0, NBT, body, 0)

    with ns("ph_batches"):
        for_each_batch(do_batch)
    with ns("ph_drain"):
        for_each_batch(drain_batch)


_mesh = plsc.VectorSubcoreMesh(core_axis_name="c", subcore_axis_name="s",
                               num_cores=NC, num_subcores=NS)

_sc_call = pl.kernel(
    _body,
    out_type=[
        jax.ShapeDtypeStruct((2 * (N + NP), 128), jnp.float32),
        jax.ShapeDtypeStruct((2, E + 2 * NP), jnp.int32),
    ],
    mesh=_mesh,
    scratch_types=[
        pltpu.VMEM((N,), jnp.float32),
        pltpu.VMEM((GPAD,), jnp.int32),
        pltpu.VMEM((2 * PW,), jnp.int32),
        pltpu.VMEM((2 * PW,), jnp.int32),
        pltpu.VMEM((2 * PW,), jnp.int32),
        pltpu.VMEM((EW,), jnp.int32),
        pltpu.VMEM((2 * FB, 128), jnp.float32),
        pltpu.VMEM((2 * FB, 128), jnp.float32),
        pltpu.SemaphoreType.DMA,
        pltpu.SemaphoreType.DMA,
    ],
    compiler_params=pltpu.CompilerParams(use_tc_tiling_on_sc=False,
                                         needs_layout_passes=False),
)


@jax.jit
def kernel(org_feats, org_edges, pred_missing, gen_feats):
    orgv = (org_feats.reshape(N // 8, 8, D // 128, 128)
            .transpose(0, 2, 1, 3).reshape(2 * N, 128))
    genq = (gen_feats.reshape(N // 8, 8, (P * D) // 128, 128)
            .transpose(0, 2, 1, 3).reshape(2 * NP, 128))
    outt, fill_edges = _sc_call(orgv, org_edges, pred_missing, genq)
    fill_feats = (outt.reshape((N + NP) // 8, 2, 8, 128)
                  .transpose(0, 2, 1, 3).reshape(N + NP, D))
    return (fill_feats, fill_edges)
